# baseline (speedup 1.0000x reference)
import functools

import jax
import jax.numpy as jnp
from jax import lax
from jax.experimental import pallas as pl
from jax.experimental.pallas import tpu as pltpu
from jax.experimental.pallas import tpu_sc as plsc

N = 16384
D = 4
NUM_TILES = 16
ROWS_PER_TILE = N // NUM_TILES
ELEMS_PER_TILE = ROWS_PER_TILE * D
VECS_PER_TILE = ELEMS_PER_TILE // 16
LBL_VECS_PER_TILE = ROWS_PER_TILE // 16


def _sc_body(a_hbm, b_hbm, lbl_hbm, out_hbm,
             a_v, b_v, l_v, res_v, all_v, out_v, shared):
    sid = lax.axis_index("s")

    pltpu.sync_copy(a_hbm.at[pl.ds(sid * ELEMS_PER_TILE, ELEMS_PER_TILE)], a_v)
    pltpu.sync_copy(b_hbm.at[pl.ds(sid * ELEMS_PER_TILE, ELEMS_PER_TILE)], b_v)
    pltpu.sync_copy(lbl_hbm.at[pl.ds(sid * ROWS_PER_TILE, ROWS_PER_TILE)], l_v)

    quad = lax.broadcasted_iota(jnp.int32, (16,), 0) >> 2

    def mse_body(v, acc):
        a = a_v[pl.ds(v * 16, 16)]
        b = b_v[pl.ds(v * 16, 16)]
        d = a - b
        m = plsc.load_gather(l_v, [quad + v * 4])
        return acc + d * d * m.astype(jnp.float32)

    acc = lax.fori_loop(0, VECS_PER_TILE, mse_body,
                        jnp.zeros((16,), jnp.float32))

    def cnt_body(k, c):
        return c + l_v[pl.ds(k * 16, 16)].astype(jnp.float32)

    cnt = lax.fori_loop(0, LBL_VECS_PER_TILE, cnt_body,
                        jnp.zeros((16,), jnp.float32))

    res_v[0] = acc
    res_v[1] = cnt
    pltpu.sync_copy(res_v, shared.at[sid])
    plsc.subcore_barrier()

    @pl.when(sid == 0)
    def _():
        pltpu.sync_copy(shared, all_v)

        def red_body(i, carry):
            ts, tc = carry
            return ts + all_v[i, 0], tc + all_v[i, 1]

        ts, tc = lax.fori_loop(0, NUM_TILES, red_body,
                               (jnp.zeros((16,), jnp.float32),
                                jnp.zeros((16,), jnp.float32)))
        s_vec = jnp.full((16,), jnp.sum(ts), dtype=jnp.float32)
        c_vec = jnp.full((16,), jnp.sum(tc), dtype=jnp.float32)
        out_v[...] = s_vec / c_vec
        pltpu.sync_copy(out_v, out_hbm)


@jax.jit
def _bbox_loss(a_flat, b_flat, label):
    mesh = plsc.VectorSubcoreMesh(core_axis_name="c", subcore_axis_name="s",
                                  num_cores=1)
    call = functools.partial(
        pl.kernel,
        out_type=jax.ShapeDtypeStruct((16,), jnp.float32),
        mesh=mesh,
        compiler_params=pltpu.CompilerParams(needs_layout_passes=False),
        scratch_types=[
            pltpu.VMEM((ELEMS_PER_TILE,), jnp.float32),
            pltpu.VMEM((ELEMS_PER_TILE,), jnp.float32),
            pltpu.VMEM((ROWS_PER_TILE,), jnp.int32),
            pltpu.VMEM((2, 16), jnp.float32),
            pltpu.VMEM((NUM_TILES, 2, 16), jnp.float32),
            pltpu.VMEM((16,), jnp.float32),
            pltpu.VMEM_SHARED((NUM_TILES, 2, 16), jnp.float32),
        ],
    )(_sc_body)
    out = call(a_flat, b_flat, label)
    return out[0]


def kernel(bbox_out, bbox_target, label):
    a_flat = bbox_out.reshape(-1)
    b_flat = bbox_target.reshape(-1)
    return _bbox_loss(a_flat, b_flat, label)
